# baseline (speedup 1.0000x reference)
import functools

import jax
import jax.numpy as jnp
from jax import lax
from jax.experimental import pallas as pl
from jax.experimental.pallas import tpu as pltpu
from jax.experimental.pallas import tpu_sc as plsc

EMB = 128
B_ROWS = 4096
SEQ = 50
TOT = B_ROWS * SEQ
NW = 32
PER_W = TOT // NW
CHUNK = 64
NCH = PER_W // CHUNK
NBUF = 8
DEPTH = 4

_mesh = plsc.VectorSubcoreMesh(core_axis_name="c", subcore_axis_name="s")


@functools.partial(
    pl.kernel,
    mesh=_mesh,
    out_type=jax.ShapeDtypeStruct((TOT, EMB), jnp.float32),
    scratch_types=[
        pltpu.VMEM((PER_W,), jnp.int32),
    ] + [pltpu.VMEM((CHUNK, EMB), jnp.float32)] * NBUF
      + [pltpu.SemaphoreType.DMA] * (2 * NBUF),
)
def _emb_lookup(idx_hbm, table_hbm, out_hbm, idx_v, *bs):
    wid = lax.axis_index("s") * 2 + lax.axis_index("c")
    base = wid * PER_W

    bufs = bs[:NBUF]
    gs = bs[NBUF:2 * NBUF]
    ws = bs[2 * NBUF:]

    pltpu.sync_copy(idx_hbm.at[pl.ds(base, PER_W)], idx_v)

    def gather(j, b):
        pltpu.async_copy(
            table_hbm.at[idx_v.at[pl.ds(j * CHUNK, CHUNK)]], bufs[b], gs[b]
        )

    def gather_wait(j, b):
        pltpu.make_async_copy(
            table_hbm.at[idx_v.at[pl.ds(j * CHUNK, CHUNK)]], bufs[b], gs[b]
        ).wait()

    def write(j, b):
        pltpu.async_copy(
            bufs[b], out_hbm.at[pl.ds(base + j * CHUNK, CHUNK)], ws[b]
        )

    def write_wait(j, b):
        pltpu.make_async_copy(
            bufs[b], out_hbm.at[pl.ds(base + j * CHUNK, CHUNK)], ws[b]
        ).wait()

    for j in range(DEPTH):
        gather(j, j)

    def step(j, carry):
        for b in range(NBUF):
            jj = j * NBUF + b
            br = (b + DEPTH) % NBUF
            gather_wait(jj, b)
            write(jj, b)

            @pl.when(jj + DEPTH >= NBUF)
            def _():
                write_wait(jj + DEPTH - NBUF, br)

            @pl.when(jj + DEPTH < NCH)
            def _():
                gather(jj + DEPTH, br)

        return carry

    lax.fori_loop(0, NCH // NBUF, step, 0)

    for jj in range(NCH - NCH % NBUF, NCH):
        b = jj % NBUF
        br = (b + DEPTH) % NBUF
        gather_wait(jj, b)
        write(jj, b)
        write_wait(jj + DEPTH - NBUF, br)

    for jj in range(NCH - (NBUF - DEPTH), NCH):
        write_wait(jj, jj % NBUF)


def _mask_body(ctx_ref, out_ref):
    out_ref[...] = jnp.sign(ctx_ref[...])


_mask = pl.pallas_call(
    _mask_body,
    out_shape=jax.ShapeDtypeStruct((B_ROWS, SEQ), jnp.int32),
)


def _relayout_body(in_ref, out_ref):
    for m in range(4):
        out_ref[m] = in_ref[pl.ds(m * SEQ, SEQ), :]


_relayout = pl.pallas_call(
    _relayout_body,
    grid=(B_ROWS // 4,),
    in_specs=[pl.BlockSpec((4 * SEQ, EMB), lambda i: (i, 0))],
    out_specs=pl.BlockSpec((4, SEQ, EMB), lambda i: (i, 0, 0)),
    out_shape=jax.ShapeDtypeStruct((B_ROWS, SEQ, EMB), jnp.float32),
)


def kernel(context, table):
    ctx_flat = context.reshape(TOT)
    emb_flat = _emb_lookup(ctx_flat, table)
    return _relayout(emb_flat), _mask(context)


# speedup vs baseline: 1.5543x; 1.5543x over previous
import functools

import jax
import jax.numpy as jnp
from jax import lax
from jax.experimental import pallas as pl
from jax.experimental.pallas import tpu as pltpu
from jax.experimental.pallas import tpu_sc as plsc

EMB = 128
B_ROWS = 4096
SEQ = 50
SEQ_PAD = 56
NSPLIT = 4
B_SLICE = B_ROWS // NSPLIT
NW = 32
ROWS_W = B_SLICE // NW
NBUF = 8
DEPTH = 4

_mesh = plsc.VectorSubcoreMesh(core_axis_name="c", subcore_axis_name="s")


@functools.partial(
    pl.kernel,
    mesh=_mesh,
    out_type=jax.ShapeDtypeStruct((B_SLICE, SEQ_PAD, EMB), jnp.float32),
    scratch_types=[
        pltpu.VMEM((ROWS_W, SEQ), jnp.int32),
    ] + [pltpu.VMEM((SEQ_PAD, EMB), jnp.float32)] * NBUF
      + [pltpu.SemaphoreType.DMA] * (2 * NBUF),
)
def _emb_slice(ctx_hbm, table_hbm, out_hbm, idx_v, *bs):
    wid = lax.axis_index("s") * 2 + lax.axis_index("c")
    r0 = wid * ROWS_W

    bufs = bs[:NBUF]
    gs = bs[NBUF:2 * NBUF]
    ws = bs[2 * NBUF:]

    pltpu.sync_copy(ctx_hbm.at[pl.ds(r0, ROWS_W)], idx_v)

    def gather(j, b):
        pltpu.async_copy(
            table_hbm.at[idx_v.at[j]], bufs[b].at[pl.ds(0, SEQ)], gs[b]
        )

    def gather_wait(j, b):
        pltpu.make_async_copy(
            table_hbm.at[idx_v.at[j]], bufs[b].at[pl.ds(0, SEQ)], gs[b]
        ).wait()

    def write(j, b):
        pltpu.async_copy(bufs[b], out_hbm.at[r0 + j], ws[b])

    def write_wait(j, b):
        pltpu.make_async_copy(bufs[b], out_hbm.at[r0 + j], ws[b]).wait()

    for j in range(DEPTH):
        gather(j, j)

    def step(g, carry):
        for b in range(NBUF):
            j = g * NBUF + b
            br = (b + DEPTH) % NBUF
            gather_wait(j, b)
            write(j, b)

            @pl.when(j + DEPTH >= NBUF)
            def _():
                write_wait(j + DEPTH - NBUF, br)

            @pl.when(j + DEPTH < ROWS_W)
            def _():
                gather(j + DEPTH, br)

        return carry

    lax.fori_loop(0, ROWS_W // NBUF, step, 0)

    for j in range(ROWS_W - (NBUF - DEPTH), ROWS_W):
        write_wait(j, j % NBUF)


def _slice0_body(in_ref, out_ref):
    out_ref[...] = in_ref[:, :SEQ, :]


def _slicek_body(carry_ref, in_ref, out_ref):
    out_ref[...] = in_ref[:, :SEQ, :]


_GB = B_SLICE // 8

_slice_tc0 = pl.pallas_call(
    _slice0_body,
    grid=(_GB,),
    in_specs=[pl.BlockSpec((8, SEQ_PAD, EMB), lambda i: (i, 0, 0))],
    out_specs=pl.BlockSpec((8, SEQ, EMB), lambda i: (i, 0, 0)),
    out_shape=jax.ShapeDtypeStruct((B_ROWS, SEQ, EMB), jnp.float32),
)


def _make_slice_tc(k):
    return pl.pallas_call(
        _slicek_body,
        grid=(_GB,),
        in_specs=[
            pl.BlockSpec(memory_space=pltpu.MemorySpace.HBM),
            pl.BlockSpec((8, SEQ_PAD, EMB), lambda i: (i, 0, 0)),
        ],
        out_specs=pl.BlockSpec(
            (8, SEQ, EMB), lambda i, _k=k: (_k * _GB + i, 0, 0)
        ),
        out_shape=jax.ShapeDtypeStruct((B_ROWS, SEQ, EMB), jnp.float32),
        input_output_aliases={0: 0},
    )


_slice_tcs = [_make_slice_tc(k) for k in range(1, NSPLIT)]


def _mask_body(ctx_ref, out_ref):
    out_ref[...] = jnp.sign(ctx_ref[...])


_mask = pl.pallas_call(
    _mask_body,
    out_shape=jax.ShapeDtypeStruct((B_ROWS, SEQ), jnp.int32),
)


def kernel(context, table):
    parts = [
        _emb_slice(context[k * B_SLICE:(k + 1) * B_SLICE], table)
        for k in range(NSPLIT)
    ]
    emb = _slice_tc0(parts[0])
    for k in range(1, NSPLIT):
        emb = _slice_tcs[k - 1](emb, parts[k])
    return emb, _mask(context)


# speedup vs baseline: 4.3011x; 2.7672x over previous
import functools

import jax
import jax.numpy as jnp
from jax import lax
from jax.experimental import pallas as pl
from jax.experimental.pallas import tpu as pltpu
from jax.experimental.pallas import tpu_sc as plsc

EMB = 128
B_ROWS = 4096
SEQ = 50
NW = 32
ROWS_W = B_ROWS // NW
NBUF = 8
DEPTH = 4
NGRP = ROWS_W // NBUF

_mesh = plsc.VectorSubcoreMesh(core_axis_name="c", subcore_axis_name="s")


@functools.partial(
    pl.kernel,
    mesh=_mesh,
    compiler_params=pltpu.CompilerParams(use_tc_tiling_on_sc=True),
    out_type=jax.ShapeDtypeStruct((B_ROWS, SEQ, EMB), jnp.float32),
    scratch_types=[
        pltpu.VMEM((ROWS_W, SEQ), jnp.int32),
    ] + [pltpu.VMEM((SEQ, EMB), jnp.float32)] * NBUF
      + [pltpu.SemaphoreType.DMA] * (2 * NBUF),
)
def _emb_lookup(ctx_hbm, table_hbm, out_hbm, idx_v, *bs):
    wid = lax.axis_index("s") * 2 + lax.axis_index("c")
    r0 = wid * ROWS_W

    bufs = bs[:NBUF]
    gs = bs[NBUF:2 * NBUF]
    ws = bs[2 * NBUF:]

    pltpu.sync_copy(ctx_hbm.at[pl.ds(r0, ROWS_W)], idx_v)

    def gather(j, b):
        pltpu.async_copy(table_hbm.at[idx_v.at[j]], bufs[b], gs[b])

    def gather_wait(j, b):
        pltpu.make_async_copy(
            table_hbm.at[idx_v.at[j]], bufs[b], gs[b]
        ).wait()

    def write(j, b):
        pltpu.async_copy(bufs[b], out_hbm.at[r0 + j], ws[b])

    def write_wait(j, b):
        pltpu.make_async_copy(bufs[b], out_hbm.at[r0 + j], ws[b]).wait()

    for j in range(DEPTH):
        gather(j, j)

    def group(jj, carry):
        j0 = jj * NBUF
        for b in range(NBUF):
            j = j0 + b
            br = (b + DEPTH) % NBUF
            gather_wait(j, b)
            write(j, b)

            @pl.when(j + DEPTH >= NBUF)
            def _():
                write_wait(j + DEPTH - NBUF, br)

            @pl.when(j + DEPTH < ROWS_W)
            def _():
                gather(j + DEPTH, br)

        return carry

    lax.fori_loop(0, NGRP, group, 0)

    for j in range(ROWS_W - (NBUF - DEPTH), ROWS_W):
        write_wait(j, j % NBUF)


def _mask_body(ctx_ref, out_ref):
    out_ref[...] = jnp.sign(ctx_ref[...])


_mask = pl.pallas_call(
    _mask_body,
    out_shape=jax.ShapeDtypeStruct((B_ROWS, SEQ), jnp.int32),
)


def kernel(context, table):
    emb = _emb_lookup(context, table)
    return emb, _mask(context)
